# Initial kernel scaffold; baseline (speedup 1.0000x reference)
import jax
import jax.numpy as jnp
from jax.experimental import pallas as pl


def kernel(x, y):
    raise NotImplementedError("write your pallas kernel here")


# speedup vs baseline: 336.4163x; 336.4163x over previous
import functools

import jax
import jax.numpy as jnp
from jax import lax
from jax.experimental import pallas as pl
from jax.experimental.pallas import tpu as pltpu, tpu_sc as plsc

_LANES = 16


def _sc_interp_kernel(nv_per_w, per_w, n_minus_1, x_hbm, y_hbm, out_hbm,
                      y_v, buf_v):
    wid = lax.axis_index("s") * 2 + lax.axis_index("c")
    base = wid * per_w
    pltpu.sync_copy(y_hbm, y_v)
    pltpu.sync_copy(x_hbm.at[pl.ds(base, per_w)], buf_v)

    scale = jnp.float32(n_minus_1)
    imax = jnp.int32(n_minus_1 - 1)

    def body(i, carry):
        off = i * _LANES
        xv = buf_v[pl.ds(off, _LANES)]
        t = xv * scale
        i0 = t.astype(jnp.int32)
        i0 = jnp.minimum(jnp.maximum(i0, 0), imax)
        w = t - i0.astype(jnp.float32)
        y0 = plsc.load_gather(y_v, [i0])
        y1 = plsc.load_gather(y_v, [i0 + 1])
        buf_v[pl.ds(off, _LANES)] = y0 + w * (y1 - y0)
        return carry

    lax.fori_loop(0, nv_per_w, body, 0, unroll=4)
    pltpu.sync_copy(buf_v, out_hbm.at[pl.ds(base, per_w)])


def kernel(x, y):
    n = y.shape[0]
    total = x.size
    nw = 32
    per_w = total // nw
    assert per_w * nw == total and per_w % _LANES == 0
    nv_per_w = per_w // _LANES

    mesh = plsc.VectorSubcoreMesh(core_axis_name="c", subcore_axis_name="s")
    run = pl.kernel(
        functools.partial(_sc_interp_kernel, nv_per_w, per_w, n - 1),
        mesh=mesh,
        out_type=jax.ShapeDtypeStruct((total,), jnp.float32),
        scratch_types=[
            pltpu.VMEM((n,), jnp.float32),
            pltpu.VMEM((per_w,), jnp.float32),
        ],
        compiler_params=pltpu.CompilerParams(needs_layout_passes=False),
    )
    out = run(x.reshape(-1), y)
    return out.reshape(x.shape)


# speedup vs baseline: 466.2377x; 1.3859x over previous
import functools

import jax
import jax.numpy as jnp
from jax import lax
from jax.experimental import pallas as pl
from jax.experimental.pallas import tpu as pltpu, tpu_sc as plsc

_LANES = 16


def _sc_interp_kernel(rows_per_w, cols, n_minus_1, x_hbm, y_hbm, out_hbm,
                      y_v, buf_v):
    wid = lax.axis_index("s") * 2 + lax.axis_index("c")
    row0 = wid * rows_per_w
    pltpu.sync_copy(y_hbm, y_v)
    pltpu.sync_copy(x_hbm.at[pl.ds(row0, rows_per_w)], buf_v)

    scale = jnp.float32(n_minus_1)
    imax = jnp.int32(n_minus_1 - 1)
    offs = list(range(0, cols - _LANES, _LANES)) + [cols - _LANES]

    def body(r, carry):
        xs = [buf_v[r, pl.ds(c, _LANES)] for c in offs]
        for c, xv in zip(offs, xs):
            t = xv * scale
            i0 = t.astype(jnp.int32)
            i0 = jnp.minimum(jnp.maximum(i0, 0), imax)
            w = t - i0.astype(jnp.float32)
            y0 = plsc.load_gather(y_v, [i0])
            y1 = plsc.load_gather(y_v, [i0 + 1])
            buf_v[r, pl.ds(c, _LANES)] = y0 + w * (y1 - y0)
        return carry

    lax.fori_loop(0, rows_per_w, body, 0, unroll=2)
    pltpu.sync_copy(buf_v, out_hbm.at[pl.ds(row0, rows_per_w)])


def kernel(x, y):
    n = y.shape[0]
    rows, cols = x.shape
    nw = 32
    rows_per_w = rows // nw
    assert rows_per_w * nw == rows and cols >= _LANES

    mesh = plsc.VectorSubcoreMesh(core_axis_name="c", subcore_axis_name="s")
    run = pl.kernel(
        functools.partial(_sc_interp_kernel, rows_per_w, cols, n - 1),
        mesh=mesh,
        out_type=jax.ShapeDtypeStruct((rows, cols), jnp.float32),
        scratch_types=[
            pltpu.VMEM((n,), jnp.float32),
            pltpu.VMEM((rows_per_w, cols), jnp.float32),
        ],
        compiler_params=pltpu.CompilerParams(needs_layout_passes=False),
    )
    return run(x, y)


# speedup vs baseline: 505.0887x; 1.0833x over previous
import functools

import jax
import jax.numpy as jnp
from jax import lax
from jax.experimental import pallas as pl
from jax.experimental.pallas import tpu as pltpu, tpu_sc as plsc

_LANES = 16
_NCHUNK = 4


def _sc_interp_kernel(rows_per_w, cols, n, x_hbm, y_hbm, out_hbm,
                      y_v, dy_v, ib0, ib1, ob0, ob1, si0, si1, so0, so1):
    wid = lax.axis_index("s") * 2 + lax.axis_index("c")
    row0 = wid * rows_per_w
    crows = rows_per_w // _NCHUNK

    ibufs, obufs = (ib0, ib1), (ob0, ob1)
    isems, osems = (si0, si1), (so0, so1)

    def start_in(k):
        return pltpu.async_copy(
            x_hbm.at[pl.ds(row0 + k * crows, crows)], ibufs[k % 2],
            isems[k % 2])

    def start_out(k):
        return pltpu.async_copy(
            obufs[k % 2], out_hbm.at[pl.ds(row0 + k * crows, crows)],
            osems[k % 2])

    in_cp = {0: start_in(0), 1: start_in(1)}
    pltpu.sync_copy(y_hbm, y_v)

    for j in range((n - 1) // _LANES):
        v = y_v[pl.ds(j * _LANES, _LANES)]
        vn = y_v[pl.ds(j * _LANES + 1, _LANES)]
        dy_v[pl.ds(j * _LANES, _LANES)] = vn - v

    scale = jnp.float32(n - 1)
    offs = list(range(0, cols - _LANES, _LANES)) + [cols - _LANES]
    out_cp = {}

    for k in range(_NCHUNK):
        ib, ob = ibufs[k % 2], obufs[k % 2]
        in_cp.pop(k).wait()
        if k >= 2:
            out_cp.pop(k - 2).wait()

        def body(r, carry, ib=ib, ob=ob):
            xs = [ib[r, pl.ds(c, _LANES)] for c in offs]
            for c, xv in zip(offs, xs):
                t = xv * scale
                i0 = t.astype(jnp.int32)
                w = t - i0.astype(jnp.float32)
                y0 = plsc.load_gather(y_v, [i0])
                d0 = plsc.load_gather(dy_v, [i0])
                ob[r, pl.ds(c, _LANES)] = y0 + w * d0
            return carry

        lax.fori_loop(0, crows, body, 0, unroll=4)
        out_cp[k] = start_out(k)
        if k + 2 < _NCHUNK:
            in_cp[k + 2] = start_in(k + 2)

    for k in sorted(out_cp):
        out_cp[k].wait()


def kernel(x, y):
    n = y.shape[0]
    rows, cols = x.shape
    nw = 32
    rows_per_w = rows // nw
    assert rows_per_w * nw == rows and cols >= _LANES
    assert (n - 1) % _LANES == 0 and rows_per_w % _NCHUNK == 0
    crows = rows_per_w // _NCHUNK

    mesh = plsc.VectorSubcoreMesh(core_axis_name="c", subcore_axis_name="s")
    run = pl.kernel(
        functools.partial(_sc_interp_kernel, rows_per_w, cols, n),
        mesh=mesh,
        out_type=jax.ShapeDtypeStruct((rows, cols), jnp.float32),
        scratch_types=[
            pltpu.VMEM((n,), jnp.float32),
            pltpu.VMEM((n - 1,), jnp.float32),
            pltpu.VMEM((crows, cols), jnp.float32),
            pltpu.VMEM((crows, cols), jnp.float32),
            pltpu.VMEM((crows, cols), jnp.float32),
            pltpu.VMEM((crows, cols), jnp.float32),
            pltpu.SemaphoreType.DMA,
            pltpu.SemaphoreType.DMA,
            pltpu.SemaphoreType.DMA,
            pltpu.SemaphoreType.DMA,
        ],
        compiler_params=pltpu.CompilerParams(needs_layout_passes=False),
    )
    return run(x, y)


# speedup vs baseline: 728.1026x; 1.4415x over previous
import functools

import jax
import jax.numpy as jnp
from jax import lax
from jax.experimental import pallas as pl
from jax.experimental.pallas import tpu as pltpu, tpu_sc as plsc

_LANES = 16
_NCHUNK = 4


def _sc_interp_kernel(rows_per_w, cols, n, x_hbm, y_hbm, out_hbm,
                      y_v, dy_v, ib0, ib1, ob0, ob1, si0, si1, so0, so1):
    wid = lax.axis_index("s") * 2 + lax.axis_index("c")
    row0 = wid * rows_per_w
    crows = rows_per_w // _NCHUNK

    ibufs, obufs = (ib0, ib1), (ob0, ob1)
    isems, osems = (si0, si1), (so0, so1)

    def start_in(k):
        return pltpu.async_copy(
            x_hbm.at[pl.ds(row0 + k * crows, crows)], ibufs[k % 2],
            isems[k % 2])

    def start_out(k):
        return pltpu.async_copy(
            obufs[k % 2], out_hbm.at[pl.ds(row0 + k * crows, crows)],
            osems[k % 2])

    in_cp = {0: start_in(0), 1: start_in(1)}
    pltpu.sync_copy(y_hbm, y_v)

    for j in range((n - 1) // _LANES):
        v = y_v[pl.ds(j * _LANES, _LANES)]
        vn = y_v[pl.ds(j * _LANES + 1, _LANES)]
        dy_v[pl.ds(j * _LANES, _LANES)] = vn - v

    scale = jnp.float32(n - 1)
    offs = list(range(0, cols - _LANES, _LANES)) + [cols - _LANES]
    out_cp = {}

    for k in range(_NCHUNK):
        ib, ob = ibufs[k % 2], obufs[k % 2]
        in_cp.pop(k).wait()
        if k >= 2:
            out_cp.pop(k - 2).wait()

        @plsc.parallel_loop(0, crows, unroll=4)
        def body(r, ib=ib, ob=ob):
            xs = [ib[r, pl.ds(c, _LANES)] for c in offs]
            for c, xv in zip(offs, xs):
                t = xv * scale
                i0 = t.astype(jnp.int32)
                w = t - i0.astype(jnp.float32)
                y0 = plsc.load_gather(y_v, [i0])
                d0 = plsc.load_gather(dy_v, [i0])
                ob[r, pl.ds(c, _LANES)] = y0 + w * d0
        out_cp[k] = start_out(k)
        if k + 2 < _NCHUNK:
            in_cp[k + 2] = start_in(k + 2)

    for k in sorted(out_cp):
        out_cp[k].wait()


def kernel(x, y):
    n = y.shape[0]
    rows, cols = x.shape
    nw = 32
    rows_per_w = rows // nw
    assert rows_per_w * nw == rows and cols >= _LANES
    assert (n - 1) % _LANES == 0 and rows_per_w % _NCHUNK == 0
    crows = rows_per_w // _NCHUNK

    mesh = plsc.VectorSubcoreMesh(core_axis_name="c", subcore_axis_name="s")
    run = pl.kernel(
        functools.partial(_sc_interp_kernel, rows_per_w, cols, n),
        mesh=mesh,
        out_type=jax.ShapeDtypeStruct((rows, cols), jnp.float32),
        scratch_types=[
            pltpu.VMEM((n,), jnp.float32),
            pltpu.VMEM((n - 1,), jnp.float32),
            pltpu.VMEM((crows, cols), jnp.float32),
            pltpu.VMEM((crows, cols), jnp.float32),
            pltpu.VMEM((crows, cols), jnp.float32),
            pltpu.VMEM((crows, cols), jnp.float32),
            pltpu.SemaphoreType.DMA,
            pltpu.SemaphoreType.DMA,
            pltpu.SemaphoreType.DMA,
            pltpu.SemaphoreType.DMA,
        ],
        compiler_params=pltpu.CompilerParams(needs_layout_passes=False),
    )
    return run(x, y)
